# Initial kernel scaffold; baseline (speedup 1.0000x reference)
import jax
import jax.numpy as jnp
from jax.experimental import pallas as pl


def kernel(x, edge_index, W1, b1, W2, b2):
    raise NotImplementedError("write your pallas kernel here")


# speedup vs baseline: 30.6919x; 30.6919x over previous
import functools

import jax
import jax.numpy as jnp
from jax import lax
from jax.experimental import pallas as pl
from jax.experimental.pallas import tpu as pltpu
from jax.experimental.pallas import tpu_sc as plsc

N = 10000
E = 320000
D_IN = 128
HID = 16
D_OUT = 128

NC = 2
NS = 16
NW = NC * NS
BLK = 128
NBLK = 79
EPT = NBLK * BLK
PAD_E = NW * EPT
ROWS_PT = 632
ACC_N = ROWS_PT * NS
TROWS_PT = N // NS


def _sc_pass_body(do_gather, table_hbm, srcp_hbm, dstp_hbm, out_hbm,
                  acc_sh, src_v, dst_v, rows_v, stage_v, sem0):
  cid = lax.axis_index("c")
  sid = lax.axis_index("s")
  wid = cid * NS + sid

  def zrow(i, _):
    stage_v[i, :] = jnp.zeros((HID,), jnp.float32)
    return 0
  lax.fori_loop(0, ROWS_PT, zrow, 0)
  pltpu.sync_copy(stage_v, acc_sh.at[pl.ds(sid * ROWS_PT, ROWS_PT)])

  if not do_gather:
    def orow(i, _):
      rows_v[i, :] = jnp.ones((HID,), jnp.float32)
      return 0
    lax.fori_loop(0, BLK, orow, 0)

  plsc.subcore_barrier()

  if do_gather:
    pltpu.sync_copy(srcp_hbm.at[wid], src_v)
  pltpu.sync_copy(dstp_hbm.at[wid], dst_v)

  def step(j, _):
    if do_gather:
      pltpu.async_copy(table_hbm.at[src_v.at[j]], rows_v, sem0).wait()
    pltpu.sync_copy(rows_v, acc_sh.at[dst_v.at[j]], add=True)
    return 0
  lax.fori_loop(0, NBLK, step, 0)

  plsc.subcore_barrier()

  pltpu.sync_copy(acc_sh.at[pl.ds(sid * ROWS_PT, ROWS_PT)],
                  out_hbm.at[cid, pl.ds(sid * ROWS_PT, ROWS_PT)])


def _make_sc_pass(do_gather):
  return functools.partial(
      pl.kernel,
      out_type=jax.ShapeDtypeStruct((NC, ACC_N, HID), jnp.float32),
      mesh=plsc.VectorSubcoreMesh(core_axis_name="c", subcore_axis_name="s"),
      compiler_params=pltpu.CompilerParams(use_tc_tiling_on_sc=False),
      scratch_types=[
          pltpu.VMEM_SHARED((ACC_N, HID), jnp.float32),
          pltpu.VMEM((NBLK, BLK), jnp.int32),
          pltpu.VMEM((NBLK, BLK), jnp.int32),
          pltpu.VMEM((BLK, HID), jnp.float32),
          pltpu.VMEM((ROWS_PT, HID), jnp.float32),
          pltpu.SemaphoreType.DMA,
      ],
  )(functools.partial(_sc_pass_body, do_gather))


_sc_gather_pass = _make_sc_pass(True)
_sc_ones_pass = _make_sc_pass(False)


def _stage_a_body(x_ref, w1_ref, p0_ref, p1_ref, z1_ref, t1_ref, dism_ref):
  deg = p0_ref[...] + p1_ref[...] + 1.0
  dism = lax.rsqrt(deg)
  z1 = jnp.dot(x_ref[...], w1_ref[...], preferred_element_type=jnp.float32)
  z1_ref[...] = z1
  dism_ref[...] = dism
  t1_ref[...] = z1 * dism


_stage_a = pl.pallas_call(
    _stage_a_body,
    out_shape=(
        jax.ShapeDtypeStruct((N, HID), jnp.float32),
        jax.ShapeDtypeStruct((N, HID), jnp.float32),
        jax.ShapeDtypeStruct((N, HID), jnp.float32),
    ),
)


def _stage_b_body(p0_ref, p1_ref, z1_ref, dism_ref, b1_ref, h_ref, t2_ref):
  dism = dism_ref[...]
  pre = dism * (p0_ref[...] + p1_ref[...]) + dism * dism * z1_ref[...] \
      + b1_ref[...]
  h = jnp.maximum(pre, 0.0)
  h_ref[...] = h
  t2_ref[...] = h * dism


_stage_b = pl.pallas_call(
    _stage_b_body,
    out_shape=(
        jax.ShapeDtypeStruct((N, HID), jnp.float32),
        jax.ShapeDtypeStruct((N, HID), jnp.float32),
    ),
)


def _stage_c_body(p0_ref, p1_ref, h_ref, dism_ref, w2_ref, b2_ref, out_ref):
  dism = dism_ref[...]
  agg = dism * (p0_ref[...] + p1_ref[...]) + dism * dism * h_ref[...]
  out_ref[...] = jnp.dot(agg, w2_ref[...],
                         preferred_element_type=jnp.float32) + b2_ref[...]


_stage_c = pl.pallas_call(
    _stage_c_body,
    out_shape=jax.ShapeDtypeStruct((N, D_OUT), jnp.float32),
)


def kernel(x, edge_index, W1, b1, W2, b2):
  src = edge_index[0].astype(jnp.int32)
  dst = edge_index[1].astype(jnp.int32)
  npad = PAD_E - E
  src_p = jnp.concatenate([src, jnp.zeros((npad,), jnp.int32)])
  dst_p = jnp.concatenate([dst, jnp.full((npad,), N, jnp.int32)])
  srcp = src_p.reshape(NW, NBLK, BLK)
  dstp = dst_p.reshape(NW, NBLK, BLK)

  ones_t = jnp.ones((N, HID), jnp.float32)

  degp = _sc_ones_pass(ones_t, srcp, dstp)
  z1, t1, dism = _stage_a(x, W1, degp[0, :N], degp[1, :N])
  seg1 = _sc_gather_pass(t1, srcp, dstp)
  h, t2 = _stage_b(seg1[0, :N], seg1[1, :N], z1, dism, b1.reshape(1, HID))
  seg2 = _sc_gather_pass(t2, srcp, dstp)
  out = _stage_c(seg2[0, :N], seg2[1, :N], h, dism, W2, b2.reshape(1, D_OUT))
  return out


# speedup vs baseline: 50.9390x; 1.6597x over previous
import functools

import jax
import jax.numpy as jnp
from jax import lax
from jax.experimental import pallas as pl
from jax.experimental.pallas import tpu as pltpu
from jax.experimental.pallas import tpu_sc as plsc

N = 10000
E = 320000
D_IN = 128
HID = 16
D_OUT = 128

NC = 2
NS = 16
NW = NC * NS
BLK = 128
NBLK = 79
EPT = NBLK * BLK
PAD_E = NW * EPT
ROWS_PT = 632
ACC_N = ROWS_PT * NS
TROWS_PT = N // NS


def _sc_pass_body(do_gather, table_hbm, srcp_hbm, dstp_hbm, out_hbm,
                  acc_sh, tab_sh, src_v, dst_v, rows0, rows1, stage_v,
                  sem0, sem1):
  cid = lax.axis_index("c")
  sid = lax.axis_index("s")
  wid = cid * NS + sid

  if do_gather:
    pltpu.sync_copy(table_hbm.at[pl.ds(sid * TROWS_PT, TROWS_PT)],
                    tab_sh.at[pl.ds(sid * TROWS_PT, TROWS_PT)])
    pltpu.sync_copy(srcp_hbm.at[wid], src_v)
  pltpu.sync_copy(dstp_hbm.at[wid], dst_v)

  def zrow(i, _):
    stage_v[i, :] = jnp.zeros((HID,), jnp.float32)
    return 0
  lax.fori_loop(0, ROWS_PT, zrow, 0)
  pltpu.sync_copy(stage_v, acc_sh.at[pl.ds(sid * ROWS_PT, ROWS_PT)])

  if not do_gather:
    def orow(i, _):
      rows0[i, :] = jnp.ones((HID,), jnp.float32)
      return 0
    lax.fori_loop(0, BLK, orow, 0)

  plsc.subcore_barrier()

  if do_gather:
    pltpu.async_copy(tab_sh.at[src_v.at[0]], rows0, sem0)

    def pair(k, _):
      j0 = 2 * k
      pltpu.make_async_copy(tab_sh.at[src_v.at[j0]], rows0, sem0).wait()
      pltpu.async_copy(tab_sh.at[src_v.at[j0 + 1]], rows1, sem1)
      pltpu.sync_copy(rows0, acc_sh.at[dst_v.at[j0]], add=True)
      pltpu.make_async_copy(tab_sh.at[src_v.at[j0 + 1]], rows1, sem1).wait()
      pltpu.async_copy(tab_sh.at[src_v.at[j0 + 2]], rows0, sem0)
      pltpu.sync_copy(rows1, acc_sh.at[dst_v.at[j0 + 1]], add=True)
      return 0
    lax.fori_loop(0, (NBLK - 1) // 2, pair, 0)
    pltpu.make_async_copy(tab_sh.at[src_v.at[NBLK - 1]], rows0, sem0).wait()
    pltpu.sync_copy(rows0, acc_sh.at[dst_v.at[NBLK - 1]], add=True)
  else:
    def step(j, _):
      pltpu.sync_copy(rows0, acc_sh.at[dst_v.at[j]], add=True)
      return 0
    lax.fori_loop(0, NBLK, step, 0)

  plsc.subcore_barrier()

  pltpu.sync_copy(acc_sh.at[pl.ds(sid * ROWS_PT, ROWS_PT)],
                  out_hbm.at[cid, pl.ds(sid * ROWS_PT, ROWS_PT)])


def _make_sc_pass(do_gather):
  return functools.partial(
      pl.kernel,
      out_type=jax.ShapeDtypeStruct((NC, ACC_N, HID), jnp.float32),
      mesh=plsc.VectorSubcoreMesh(core_axis_name="c", subcore_axis_name="s"),
      compiler_params=pltpu.CompilerParams(use_tc_tiling_on_sc=False),
      scratch_types=[
          pltpu.VMEM_SHARED((ACC_N, HID), jnp.float32),
          pltpu.VMEM_SHARED((N, HID), jnp.float32),
          pltpu.VMEM((NBLK, BLK), jnp.int32),
          pltpu.VMEM((NBLK, BLK), jnp.int32),
          pltpu.VMEM((BLK, HID), jnp.float32),
          pltpu.VMEM((BLK, HID), jnp.float32),
          pltpu.VMEM((ROWS_PT, HID), jnp.float32),
          pltpu.SemaphoreType.DMA,
          pltpu.SemaphoreType.DMA,
      ],
  )(functools.partial(_sc_pass_body, do_gather))


_sc_gather_pass = _make_sc_pass(True)
_sc_ones_pass = _make_sc_pass(False)


def _stage_a_body(x_ref, w1_ref, p0_ref, p1_ref, z1_ref, t1_ref, dism_ref):
  deg = p0_ref[...] + p1_ref[...] + 1.0
  dism = lax.rsqrt(deg)
  z1 = jnp.dot(x_ref[...], w1_ref[...], preferred_element_type=jnp.float32)
  z1_ref[...] = z1
  dism_ref[...] = dism
  t1_ref[...] = z1 * dism


_stage_a = pl.pallas_call(
    _stage_a_body,
    out_shape=(
        jax.ShapeDtypeStruct((N, HID), jnp.float32),
        jax.ShapeDtypeStruct((N, HID), jnp.float32),
        jax.ShapeDtypeStruct((N, HID), jnp.float32),
    ),
)


def _stage_b_body(p0_ref, p1_ref, z1_ref, dism_ref, b1_ref, h_ref, t2_ref):
  dism = dism_ref[...]
  pre = dism * (p0_ref[...] + p1_ref[...]) + dism * dism * z1_ref[...] \
      + b1_ref[...]
  h = jnp.maximum(pre, 0.0)
  h_ref[...] = h
  t2_ref[...] = h * dism


_stage_b = pl.pallas_call(
    _stage_b_body,
    out_shape=(
        jax.ShapeDtypeStruct((N, HID), jnp.float32),
        jax.ShapeDtypeStruct((N, HID), jnp.float32),
    ),
)


def _stage_c_body(p0_ref, p1_ref, h_ref, dism_ref, w2_ref, b2_ref, out_ref):
  dism = dism_ref[...]
  agg = dism * (p0_ref[...] + p1_ref[...]) + dism * dism * h_ref[...]
  out_ref[...] = jnp.dot(agg, w2_ref[...],
                         preferred_element_type=jnp.float32) + b2_ref[...]


_stage_c = pl.pallas_call(
    _stage_c_body,
    out_shape=jax.ShapeDtypeStruct((N, D_OUT), jnp.float32),
)


def kernel(x, edge_index, W1, b1, W2, b2):
  src = edge_index[0].astype(jnp.int32)
  dst = edge_index[1].astype(jnp.int32)
  npad = PAD_E - E
  src_p = jnp.concatenate([src, jnp.zeros((npad,), jnp.int32)])
  dst_p = jnp.concatenate([dst, jnp.full((npad,), N, jnp.int32)])
  srcp = src_p.reshape(NW, NBLK, BLK)
  dstp = dst_p.reshape(NW, NBLK, BLK)

  ones_t = jnp.ones((N, HID), jnp.float32)

  degp = _sc_ones_pass(ones_t, srcp, dstp)
  z1, t1, dism = _stage_a(x, W1, degp[0, :N], degp[1, :N])
  seg1 = _sc_gather_pass(t1, srcp, dstp)
  h, t2 = _stage_b(seg1[0, :N], seg1[1, :N], z1, dism, b1.reshape(1, HID))
  seg2 = _sc_gather_pass(t2, srcp, dstp)
  out = _stage_c(seg2[0, :N], seg2[1, :N], h, dism, W2, b2.reshape(1, D_OUT))
  return out


# speedup vs baseline: 56.8839x; 1.1167x over previous
import functools

import jax
import jax.numpy as jnp
from jax import lax
from jax.experimental import pallas as pl
from jax.experimental.pallas import tpu as pltpu
from jax.experimental.pallas import tpu_sc as plsc

N = 10000
E = 320000
D_IN = 128
HID = 16
D_OUT = 128

NC = 2
NS = 16
NW = NC * NS
BLK = 128
NBLK = 79
EPT = NBLK * BLK
PAD_E = NW * EPT
ROWS_PT = 632
ACC_N = ROWS_PT * NS


def _sc_pass_body(do_gather, table_hbm, srcp_hbm, dstp_hbm, out_hbm,
                  acc_sh, tab_sh, src_v, dst_v, rows0, rows1, stage_v,
                  sem0, sem1):
  cid = lax.axis_index("c")
  sid = lax.axis_index("s")
  wid = cid * NS + sid

  if do_gather:
    pltpu.sync_copy(table_hbm.at[pl.ds(sid * ROWS_PT, ROWS_PT)],
                    tab_sh.at[pl.ds(sid * ROWS_PT, ROWS_PT)])
    pltpu.sync_copy(srcp_hbm.at[wid], src_v)
  pltpu.sync_copy(dstp_hbm.at[wid], dst_v)

  def zrow(i, _):
    stage_v[i, :] = jnp.zeros((HID,), jnp.float32)
    return 0
  lax.fori_loop(0, ROWS_PT, zrow, 0)
  pltpu.sync_copy(stage_v, acc_sh.at[pl.ds(sid * ROWS_PT, ROWS_PT)])

  if not do_gather:
    def orow(i, _):
      rows0[i, :] = jnp.ones((HID,), jnp.float32)
      return 0
    lax.fori_loop(0, BLK, orow, 0)

  plsc.subcore_barrier()

  if do_gather:
    pltpu.async_copy(tab_sh.at[src_v.at[0]], rows0, sem0)

    def pair(k, _):
      j0 = 2 * k
      pltpu.make_async_copy(tab_sh.at[src_v.at[j0]], rows0, sem0).wait()
      pltpu.async_copy(tab_sh.at[src_v.at[j0 + 1]], rows1, sem1)
      pltpu.sync_copy(rows0, acc_sh.at[dst_v.at[j0]], add=True)
      pltpu.make_async_copy(tab_sh.at[src_v.at[j0 + 1]], rows1, sem1).wait()
      pltpu.async_copy(tab_sh.at[src_v.at[j0 + 2]], rows0, sem0)
      pltpu.sync_copy(rows1, acc_sh.at[dst_v.at[j0 + 1]], add=True)
      return 0
    lax.fori_loop(0, (NBLK - 1) // 2, pair, 0)
    pltpu.make_async_copy(tab_sh.at[src_v.at[NBLK - 1]], rows0, sem0).wait()
    pltpu.sync_copy(rows0, acc_sh.at[dst_v.at[NBLK - 1]], add=True)
  else:
    def step(j, _):
      pltpu.sync_copy(rows0, acc_sh.at[dst_v.at[j]], add=True)
      return 0
    lax.fori_loop(0, NBLK, step, 0)

  plsc.subcore_barrier()

  pltpu.sync_copy(acc_sh.at[pl.ds(sid * ROWS_PT, ROWS_PT)],
                  out_hbm.at[cid, pl.ds(sid * ROWS_PT, ROWS_PT)])


def _make_sc_pass(do_gather):
  return functools.partial(
      pl.kernel,
      out_type=jax.ShapeDtypeStruct((NC, ACC_N, HID), jnp.float32),
      mesh=plsc.VectorSubcoreMesh(core_axis_name="c", subcore_axis_name="s"),
      compiler_params=pltpu.CompilerParams(use_tc_tiling_on_sc=False),
      scratch_types=[
          pltpu.VMEM_SHARED((ACC_N, HID), jnp.float32),
          pltpu.VMEM_SHARED((ACC_N, HID), jnp.float32),
          pltpu.VMEM((NBLK, BLK), jnp.int32),
          pltpu.VMEM((NBLK, BLK), jnp.int32),
          pltpu.VMEM((BLK, HID), jnp.float32),
          pltpu.VMEM((BLK, HID), jnp.float32),
          pltpu.VMEM((ROWS_PT, HID), jnp.float32),
          pltpu.SemaphoreType.DMA,
          pltpu.SemaphoreType.DMA,
      ],
  )(functools.partial(_sc_pass_body, do_gather))


_sc_gather_pass = _make_sc_pass(True)
_sc_ones_pass = _make_sc_pass(False)


def _stage_a_body(x_ref, w1_ref, degp_ref, z1_ref, t1_ref, dism_ref):
  deg = degp_ref[0] + degp_ref[1] + 1.0
  dism = lax.rsqrt(deg)
  z1 = jnp.dot(x_ref[...], w1_ref[...], preferred_element_type=jnp.float32)
  z1 = jnp.concatenate(
      [z1, jnp.zeros((ACC_N - N, HID), jnp.float32)], axis=0)
  z1_ref[...] = z1
  dism_ref[...] = dism
  t1_ref[...] = z1 * dism


_stage_a = pl.pallas_call(
    _stage_a_body,
    out_shape=(
        jax.ShapeDtypeStruct((ACC_N, HID), jnp.float32),
        jax.ShapeDtypeStruct((ACC_N, HID), jnp.float32),
        jax.ShapeDtypeStruct((ACC_N, HID), jnp.float32),
    ),
)


def _stage_b_body(segp_ref, z1_ref, dism_ref, b1_ref, h_ref, t2_ref):
  dism = dism_ref[...]
  pre = dism * (segp_ref[0] + segp_ref[1]) + dism * dism * z1_ref[...] \
      + b1_ref[...]
  h = jnp.maximum(pre, 0.0)
  h_ref[...] = h
  t2_ref[...] = h * dism


_stage_b = pl.pallas_call(
    _stage_b_body,
    out_shape=(
        jax.ShapeDtypeStruct((ACC_N, HID), jnp.float32),
        jax.ShapeDtypeStruct((ACC_N, HID), jnp.float32),
    ),
)


def _stage_c_body(segp_ref, h_ref, dism_ref, w2_ref, b2_ref, out_ref):
  dism = dism_ref[...]
  agg = dism * (segp_ref[0] + segp_ref[1]) + dism * dism * h_ref[...]
  out_ref[...] = jnp.dot(agg[:N], w2_ref[...],
                         preferred_element_type=jnp.float32) + b2_ref[...]


_stage_c = pl.pallas_call(
    _stage_c_body,
    out_shape=jax.ShapeDtypeStruct((N, D_OUT), jnp.float32),
)


def kernel(x, edge_index, W1, b1, W2, b2):
  src = edge_index[0].astype(jnp.int32)
  dst = edge_index[1].astype(jnp.int32)
  npad = PAD_E - E
  src_p = jnp.concatenate([src, jnp.zeros((npad,), jnp.int32)])
  dst_p = jnp.concatenate([dst, jnp.full((npad,), N, jnp.int32)])
  srcp = src_p.reshape(NW, NBLK, BLK)
  dstp = dst_p.reshape(NW, NBLK, BLK)

  ones_t = jnp.ones((ACC_N, HID), jnp.float32)

  degp = _sc_ones_pass(ones_t, srcp, dstp)
  z1, t1, dism = _stage_a(x, W1, degp)
  seg1 = _sc_gather_pass(t1, srcp, dstp)
  h, t2 = _stage_b(seg1, z1, dism, b1.reshape(1, HID))
  seg2 = _sc_gather_pass(t2, srcp, dstp)
  out = _stage_c(seg2, h, dism, W2, b2.reshape(1, D_OUT))
  return out


# speedup vs baseline: 65.5387x; 1.1521x over previous
import functools

import jax
import jax.numpy as jnp
from jax import lax
from jax.experimental import pallas as pl
from jax.experimental.pallas import tpu as pltpu
from jax.experimental.pallas import tpu_sc as plsc

N = 10000
E = 320000
D_IN = 128
HID = 16
D_OUT = 128

NC = 2
NS = 16
NW = NC * NS
BLK = 128
NBLK = 79
EPT = NBLK * BLK
PAD_E = NW * EPT
ROWS_PT = 632
ACC_N = ROWS_PT * NS

_MESH = plsc.VectorSubcoreMesh(core_axis_name="c", subcore_axis_name="s")
_PARAMS = pltpu.CompilerParams(use_tc_tiling_on_sc=False, needs_layout_passes=False)

_NODE = jax.ShapeDtypeStruct((ACC_N, HID), jnp.float32)
_PART = jax.ShapeDtypeStruct((NC, ACC_N, HID), jnp.float32)


def _rsqrt_nr(d):
  i = plsc.bitcast(d, jnp.int32)
  i = jnp.int32(0x5F3759DF) - lax.shift_right_logical(i, 1)
  y = plsc.bitcast(i, jnp.float32)
  for _ in range(3):
    y = y * (1.5 - 0.5 * d * y * y)
  return y


def _zero_fill(buf, nrows):
  def zrow(i, _):
    buf[i, :] = jnp.zeros((HID,), jnp.float32)
    return 0
  lax.fori_loop(0, nrows, zrow, 0)


def _edge_pass(tab_sh, acc_sh, src_v, dst_v, rows0, rows1, sem0, sem1):
  pltpu.async_copy(tab_sh.at[src_v.at[0]], rows0, sem0)

  def pair(k, _):
    j0 = 2 * k
    pltpu.make_async_copy(tab_sh.at[src_v.at[j0]], rows0, sem0).wait()
    pltpu.async_copy(tab_sh.at[src_v.at[j0 + 1]], rows1, sem1)
    pltpu.sync_copy(rows0, acc_sh.at[dst_v.at[j0]], add=True)
    pltpu.make_async_copy(tab_sh.at[src_v.at[j0 + 1]], rows1, sem1).wait()
    pltpu.async_copy(tab_sh.at[src_v.at[j0 + 2]], rows0, sem0)
    pltpu.sync_copy(rows1, acc_sh.at[dst_v.at[j0 + 1]], add=True)
    return 0
  lax.fori_loop(0, (NBLK - 1) // 2, pair, 0)
  pltpu.make_async_copy(tab_sh.at[src_v.at[NBLK - 1]], rows0, sem0).wait()
  pltpu.sync_copy(rows0, acc_sh.at[dst_v.at[NBLK - 1]], add=True)


def _writeout(acc_sh, out_hbm, cid, sid):
  pltpu.sync_copy(acc_sh.at[pl.ds(sid * ROWS_PT, ROWS_PT)],
                  out_hbm.at[cid, pl.ds(sid * ROWS_PT, ROWS_PT)])


def _sc_deg_body(dstp_hbm, out_hbm, acc_sh, dst_v, rows0, stage_v, sem0):
  cid = lax.axis_index("c")
  sid = lax.axis_index("s")
  wid = cid * NS + sid

  pltpu.sync_copy(dstp_hbm.at[wid], dst_v)
  _zero_fill(stage_v, ROWS_PT)
  pltpu.sync_copy(stage_v, acc_sh.at[pl.ds(sid * ROWS_PT, ROWS_PT)])

  def orow(i, _):
    rows0[i, :] = jnp.ones((HID,), jnp.float32)
    return 0
  lax.fori_loop(0, BLK, orow, 0)

  plsc.subcore_barrier()

  def step(j, _):
    pltpu.sync_copy(rows0, acc_sh.at[dst_v.at[j]], add=True)
    return 0
  lax.fori_loop(0, NBLK, step, 0)

  plsc.subcore_barrier()
  _writeout(acc_sh, out_hbm, cid, sid)


_sc_deg = pl.kernel(
    _sc_deg_body,
    out_type=_PART,
    mesh=_MESH,
    compiler_params=_PARAMS,
    scratch_types=[
        pltpu.VMEM_SHARED((ACC_N, HID), jnp.float32),
        pltpu.VMEM((NBLK, BLK), jnp.int32),
        pltpu.VMEM((BLK, HID), jnp.float32),
        pltpu.VMEM((ROWS_PT, HID), jnp.float32),
        pltpu.SemaphoreType.DMA,
    ],
)


def _sc_l1_body(degp_hbm, z1_hbm, srcp_hbm, dstp_hbm, seg_out, dism_out,
                acc_sh, tab_sh, src_v, dst_v, rows0, rows1, va, vb, vc,
                sem0, sem1):
  cid = lax.axis_index("c")
  sid = lax.axis_index("s")
  wid = cid * NS + sid
  sl = pl.ds(sid * ROWS_PT, ROWS_PT)

  pltpu.sync_copy(srcp_hbm.at[wid], src_v)
  pltpu.sync_copy(dstp_hbm.at[wid], dst_v)
  pltpu.sync_copy(degp_hbm.at[0, sl], va)
  pltpu.sync_copy(degp_hbm.at[1, sl], vb)
  pltpu.sync_copy(z1_hbm.at[sl], vc)

  def prep(r, _):
    deg = va[r, :] + vb[r, :] + 1.0
    y = _rsqrt_nr(deg)
    va[r, :] = vc[r, :] * y
    vb[r, :] = y
    return 0
  lax.fori_loop(0, ROWS_PT, prep, 0)

  pltpu.sync_copy(va, tab_sh.at[sl])

  @pl.when(cid == 0)
  def _():
    pltpu.sync_copy(va, acc_sh.at[sl])
    pltpu.sync_copy(vb, dism_out.at[sl])

  @pl.when(cid != 0)
  def _():
    _zero_fill(vc, ROWS_PT)
    pltpu.sync_copy(vc, acc_sh.at[sl])

  plsc.subcore_barrier()
  _edge_pass(tab_sh, acc_sh, src_v, dst_v, rows0, rows1, sem0, sem1)
  plsc.subcore_barrier()
  _writeout(acc_sh, seg_out, cid, sid)


_sc_l1 = pl.kernel(
    _sc_l1_body,
    out_type=(_PART, _NODE),
    mesh=_MESH,
    compiler_params=_PARAMS,
    scratch_types=[
        pltpu.VMEM_SHARED((ACC_N, HID), jnp.float32),
        pltpu.VMEM_SHARED((ACC_N, HID), jnp.float32),
        pltpu.VMEM((NBLK, BLK), jnp.int32),
        pltpu.VMEM((NBLK, BLK), jnp.int32),
        pltpu.VMEM((BLK, HID), jnp.float32),
        pltpu.VMEM((BLK, HID), jnp.float32),
        pltpu.VMEM((ROWS_PT, HID), jnp.float32),
        pltpu.VMEM((ROWS_PT, HID), jnp.float32),
        pltpu.VMEM((ROWS_PT, HID), jnp.float32),
        pltpu.SemaphoreType.DMA,
        pltpu.SemaphoreType.DMA,
    ],
)


def _sc_l2_body(segp_hbm, dism_hbm, b1_hbm, srcp_hbm, dstp_hbm, seg_out,
                acc_sh, tab_sh, src_v, dst_v, rows0, rows1, va, vb, vc,
                b1_v, sem0, sem1):
  cid = lax.axis_index("c")
  sid = lax.axis_index("s")
  wid = cid * NS + sid
  sl = pl.ds(sid * ROWS_PT, ROWS_PT)

  pltpu.sync_copy(srcp_hbm.at[wid], src_v)
  pltpu.sync_copy(dstp_hbm.at[wid], dst_v)
  pltpu.sync_copy(segp_hbm.at[0, sl], va)
  pltpu.sync_copy(segp_hbm.at[1, sl], vb)
  pltpu.sync_copy(dism_hbm.at[sl], vc)
  pltpu.sync_copy(b1_hbm, b1_v)

  def prep(r, _):
    y = vc[r, :]
    h = jnp.maximum(y * (va[r, :] + vb[r, :]) + b1_v[...], 0.0)
    va[r, :] = h * y
    return 0
  lax.fori_loop(0, ROWS_PT, prep, 0)

  pltpu.sync_copy(va, tab_sh.at[sl])

  @pl.when(cid == 0)
  def _():
    pltpu.sync_copy(va, acc_sh.at[sl])

  @pl.when(cid != 0)
  def _():
    _zero_fill(vc, ROWS_PT)
    pltpu.sync_copy(vc, acc_sh.at[sl])

  plsc.subcore_barrier()
  _edge_pass(tab_sh, acc_sh, src_v, dst_v, rows0, rows1, sem0, sem1)
  plsc.subcore_barrier()
  _writeout(acc_sh, seg_out, cid, sid)


_sc_l2 = pl.kernel(
    _sc_l2_body,
    out_type=_PART,
    mesh=_MESH,
    compiler_params=_PARAMS,
    scratch_types=[
        pltpu.VMEM_SHARED((ACC_N, HID), jnp.float32),
        pltpu.VMEM_SHARED((ACC_N, HID), jnp.float32),
        pltpu.VMEM((NBLK, BLK), jnp.int32),
        pltpu.VMEM((NBLK, BLK), jnp.int32),
        pltpu.VMEM((BLK, HID), jnp.float32),
        pltpu.VMEM((BLK, HID), jnp.float32),
        pltpu.VMEM((ROWS_PT, HID), jnp.float32),
        pltpu.VMEM((ROWS_PT, HID), jnp.float32),
        pltpu.VMEM((ROWS_PT, HID), jnp.float32),
        pltpu.VMEM((HID,), jnp.float32),
        pltpu.SemaphoreType.DMA,
        pltpu.SemaphoreType.DMA,
    ],
)


def _tc_in_body(x_ref, w1_ref, z1_ref):
  z1 = jnp.dot(x_ref[...], w1_ref[...], preferred_element_type=jnp.float32)
  z1_ref[...] = jnp.concatenate(
      [z1, jnp.zeros((ACC_N - N, HID), jnp.float32)], axis=0)


_tc_in = pl.pallas_call(_tc_in_body, out_shape=_NODE)


def _tc_out_body(segp_ref, dism_ref, w2_ref, b2_ref, out_ref):
  agg = dism_ref[...] * (segp_ref[0] + segp_ref[1])
  out_ref[...] = jnp.dot(agg[:N], w2_ref[...],
                         preferred_element_type=jnp.float32) + b2_ref[...]


_tc_out = pl.pallas_call(
    _tc_out_body, out_shape=jax.ShapeDtypeStruct((N, D_OUT), jnp.float32))


def kernel(x, edge_index, W1, b1, W2, b2):
  src = edge_index[0].astype(jnp.int32)
  dst = edge_index[1].astype(jnp.int32)
  npad = PAD_E - E
  src_p = jnp.concatenate([src, jnp.zeros((npad,), jnp.int32)])
  dst_p = jnp.concatenate([dst, jnp.full((npad,), N, jnp.int32)])
  srcp = src_p.reshape(NW, NBLK, BLK)
  dstp = dst_p.reshape(NW, NBLK, BLK)

  degp = _sc_deg(dstp)
  z1 = _tc_in(x, W1)
  seg1p, dism = _sc_l1(degp, z1, srcp, dstp)
  seg2p = _sc_l2(seg1p, dism, b1, srcp, dstp)
  out = _tc_out(seg2p, dism, W2, b2.reshape(1, D_OUT))
  return out
